# baseline (speedup 1.0000x reference)
import functools

import jax
import jax.numpy as jnp
from jax import lax
from jax.experimental import pallas as pl
from jax.experimental.pallas import tpu as pltpu
from jax.experimental.pallas import tpu_sc as plsc

H = 128
NL = 4
B = 256
N = 10000
E = 320000

NCORES = 2
NSUB = 16
CW = H // NCORES
CHUNK = 128
G = 2
NBUF = 6
TCH = 162
NR = TCH // G
UNROLL = 3
NCHUNKS_P = NSUB * TCH
EP = NCHUNKS_P * CHUNK
NP = 10240
RPT = NP // NSUB

_sc_mesh = plsc.VectorSubcoreMesh(core_axis_name="c", subcore_axis_name="s")


@functools.partial(
    pl.kernel,
    mesh=_sc_mesh,
    out_type=jax.ShapeDtypeStruct((NCORES, NP, CW), jnp.float32),
    scratch_types=[
        pltpu.VMEM((TCH, CHUNK), jnp.int32),
        pltpu.VMEM((NBUF, CHUNK), jnp.int32),
        pltpu.VMEM((NBUF, CHUNK), jnp.int32),
        pltpu.VMEM((NBUF, CHUNK, CW), jnp.float32),
        pltpu.VMEM_SHARED((NP, CW), jnp.float32),
        pltpu.SemaphoreType.DMA,
        pltpu.SemaphoreType.DMA,
    ],
    compiler_params=pltpu.CompilerParams(use_tc_tiling_on_sc=False),
)
def _sc_agg(x_hbm, pk_hbm, out_hbm, pk, sidx, didx, rows, acc, gsem, ssem):
    c = lax.axis_index("c")
    s = lax.axis_index("s")
    xc = x_hbm.at[c]

    zv = jnp.zeros((16,), jnp.float32)

    def _zrow(i, carry):
        for cb in range(CW // 16):
            rows[0, i, pl.ds(cb * 16, 16)] = zv
        return carry

    lax.fori_loop(0, CHUNK, _zrow, 0)
    for k in range(RPT // CHUNK):
        pltpu.sync_copy(rows.at[0], acc.at[pl.ds(s * RPT + k * CHUNK, CHUNK)])

    pltpu.sync_copy(pk_hbm.at[pl.ds(s * TCH, TCH)], pk)

    def _unpack(j, slot):
        def _u16(i, carry):
            w = pk[j, pl.ds(i * 16, 16)]
            sidx[slot, pl.ds(i * 16, 16)] = jnp.bitwise_and(w, 0xFFFF)
            didx[slot, pl.ds(i * 16, 16)] = lax.shift_right_logical(w, 16)
            return carry

        lax.fori_loop(0, CHUNK // 16, _u16, 0)

    for j in range(2 * G):
        _unpack(j, j)
        pltpu.async_copy(xc.at[sidx.at[j]], rows.at[j], gsem)
    plsc.subcore_barrier()

    def _gwait(j, slot):
        pltpu.make_async_copy(xc.at[sidx.at[slot]], rows.at[slot], gsem).wait()

    def _swait(j, slot):
        pltpu.make_async_copy(rows.at[slot], acc.at[didx.at[slot]], ssem).wait()

    def _outer(k, carry):
        for u in range(UNROLL):
            r = k * UNROLL + u
            s0 = (G * u) % NBUF
            for b in range(G):
                slot = s0 + b
                j = r * G + b
                _gwait(j, slot)
                pltpu.async_copy(rows.at[slot], acc.at[didx.at[slot]],
                                 ssem, add=True)
            p0 = (s0 - G) % NBUF
            for b in range(G):
                slot = p0 + b
                if u == 0:
                    @pl.when(k > 0)
                    def _():
                        _swait(0, slot)
                else:
                    _swait(0, slot)
                j_new = (r + 2) * G + b

                @pl.when(r < NR - 2)
                def _():
                    _unpack(j_new, slot)
                    pltpu.async_copy(xc.at[sidx.at[slot]], rows.at[slot], gsem)
        return carry

    lax.fori_loop(0, NR // UNROLL, _outer, 0)
    for b in range(G):
        slot = ((NR - 1) * G + b) % NBUF
        _swait(0, slot)
    plsc.subcore_barrier()

    pltpu.sync_copy(acc.at[pl.ds(s * RPT, RPT)], out_hbm.at[c, pl.ds(s * RPT, RPT)])


def _pad_edges(e):
    src = jnp.concatenate([e[0], jnp.zeros((EP - E,), e.dtype)])
    dst = jnp.concatenate([e[1], jnp.full((EP - E,), N, e.dtype)])
    packed = jnp.bitwise_or(jnp.left_shift(dst.astype(jnp.int32), 16),
                            src.astype(jnp.int32))
    return packed.reshape(NCHUNKS_P, CHUNK)


def _embed_body(dx_ref, w_ref, b_ref, out_ref):
    out_ref[...] = (
        jnp.dot(dx_ref[...], w_ref[...], preferred_element_type=jnp.float32)
        + b_ref[...]
    )


_embed = pl.pallas_call(
    _embed_body, out_shape=jax.ShapeDtypeStruct((N, H), jnp.float32)
)


def _bn(x, g, bt):
    mu = jnp.mean(x, axis=0, keepdims=True)
    var = jnp.mean((x - mu) ** 2, axis=0, keepdims=True)
    return g * (x - mu) * lax.rsqrt(var + 1e-5) + bt


def _layer_body(x_ref, agg_ref, eps_ref, w1_ref, b1_ref, g1_ref,
                bt1_ref, w2_ref, b2_ref, g2_ref, bt2_ref, out_ref):
    h = (1.0 + eps_ref[0, 0]) * x_ref[...] + agg_ref[...]
    t1 = jnp.dot(h, w1_ref[...], preferred_element_type=jnp.float32) + b1_ref[...]
    t1 = jax.nn.relu(_bn(t1, g1_ref[...], bt1_ref[...]))
    t2 = jnp.dot(t1, w2_ref[...], preferred_element_type=jnp.float32) + b2_ref[...]
    out_ref[...] = jax.nn.relu(_bn(t2, g2_ref[...], bt2_ref[...]))


_layer = pl.pallas_call(
    _layer_body, out_shape=jax.ShapeDtypeStruct((N, H), jnp.float32)
)


def _pool_body(x_ref, bf_ref, out_ref):
    io = lax.broadcasted_iota(jnp.int32, (B, N), 0).astype(jnp.float32)
    p = jnp.where(io == bf_ref[...], 1.0, 0.0)
    sums = jnp.dot(p, x_ref[...], preferred_element_type=jnp.float32)
    cnt = jnp.sum(p, axis=1, keepdims=True)
    out_ref[...] = sums / jnp.maximum(cnt, 1.0)


_pool = pl.pallas_call(
    _pool_body, out_shape=jax.ShapeDtypeStruct((B, H), jnp.float32)
)


def _ln(x, g, bt):
    mu = jnp.mean(x, axis=-1, keepdims=True)
    var = jnp.mean((x - mu) ** 2, axis=-1, keepdims=True)
    return g * (x - mu) * lax.rsqrt(var + 1e-5) + bt


def _fusion_body(aq_ref, bs_ref, sol_ref, t_ref,
                 tw1_ref, tb1_ref, tw2_ref, tb2_ref,
                 iwb_ref, iws_ref, ib_ref, ig_ref, ibt_ref,
                 fw1a_ref, fw1i_ref, fw1t_ref, fb1_ref, fg1_ref, fbt1_ref,
                 fw2_ref, fb2_ref, fg2_ref, fbt2_ref,
                 fw3_ref, fb3_ref, out_ref):
    inter = (
        jnp.dot(bs_ref[...], iwb_ref[...], preferred_element_type=jnp.float32)
        + jnp.dot(sol_ref[...], iws_ref[...], preferred_element_type=jnp.float32)
        + ib_ref[...]
    )
    inter = jax.nn.relu(_ln(inter, ig_ref[...], ibt_ref[...]))
    te = jax.nn.relu(t_ref[...] * tw1_ref[...] + tb1_ref[...])
    te = jnp.dot(te, tw2_ref[...], preferred_element_type=jnp.float32) + tb2_ref[...]
    h = (
        jnp.dot(aq_ref[...], fw1a_ref[...], preferred_element_type=jnp.float32)
        + jnp.dot(inter, fw1i_ref[...], preferred_element_type=jnp.float32)
        + jnp.dot(te, fw1t_ref[...], preferred_element_type=jnp.float32)
        + fb1_ref[...]
    )
    h = jax.nn.relu(_ln(h, fg1_ref[...], fbt1_ref[...]))
    h = jnp.dot(h, fw2_ref[...], preferred_element_type=jnp.float32) + fb2_ref[...]
    h = jax.nn.relu(_ln(h, fg2_ref[...], fbt2_ref[...]))
    out_ref[...] = (
        jnp.sum(h * fw3_ref[...], axis=1, keepdims=True) + fb3_ref[0, 0]
    )


_fusion = pl.pallas_call(
    _fusion_body, out_shape=jax.ShapeDtypeStruct((B, 1), jnp.float32)
)


def _encoder(x0, pk2, p):
    x = _embed(x0, p['emb_W'], p['emb_b'].reshape(1, H))
    for lp in p['layers']:
        xs = jnp.stack([x[:, :CW], x[:, CW:]])
        parts = _sc_agg(xs, pk2)
        agg = jnp.concatenate([parts[0, :N], parts[1, :N]], axis=1)
        x = _layer(
            x, agg, lp['eps'].reshape(1, 1),
            lp['W1'], lp['b1'].reshape(1, 2 * H), lp['g1'].reshape(1, 2 * H),
            lp['bt1'].reshape(1, 2 * H),
            lp['W2'], lp['b2'].reshape(1, H), lp['g2'].reshape(1, H),
            lp['bt2'].reshape(1, H),
        )
    return x


def kernel(dx, de, db, sx, se, sb, t, params):
    pk_d = _pad_edges(de)
    pk_s = _pad_edges(se)
    dbf = db.astype(jnp.float32).reshape(1, N)
    sbf = sb.astype(jnp.float32).reshape(1, N)

    x_aq = _encoder(dx, pk_d, params['aq'])
    x_bs = _encoder(dx, pk_d, params['bs'])
    x_sol = _encoder(sx, pk_s, params['sol'])

    emb_aq = _pool(x_aq, dbf)
    emb_bs = _pool(x_bs, dbf)
    emb_sol = _pool(x_sol, sbf)

    tp, ip, fp = params['temp'], params['inter'], params['fus']
    return _fusion(
        emb_aq, emb_bs, emb_sol, t,
        tp['W1'], tp['b1'].reshape(1, 32), tp['W2'], tp['b2'].reshape(1, 32),
        ip['W'][:H], ip['W'][H:], ip['b'].reshape(1, H),
        ip['g'].reshape(1, H), ip['bt'].reshape(1, H),
        fp['W1'][:H], fp['W1'][H:2 * H], fp['W1'][2 * H:],
        fp['b1'].reshape(1, H), fp['g1'].reshape(1, H), fp['bt1'].reshape(1, H),
        fp['W2'], fp['b2'].reshape(1, H // 2), fp['g2'].reshape(1, H // 2),
        fp['bt2'].reshape(1, H // 2),
        fp['W3'].reshape(1, H // 2), fp['b3'].reshape(1, 1),
    )


# speedup vs baseline: 2.4175x; 2.4175x over previous
import functools

import jax
import jax.numpy as jnp
from jax import lax
from jax.experimental import pallas as pl
from jax.experimental.pallas import tpu as pltpu
from jax.experimental.pallas import tpu_sc as plsc

H = 128
NL = 4
B = 256
N = 10000
E = 320000

NCORES = 2
NSUB = 16
CW = H // NCORES
CHUNK = 80
G = 3
NBUF = 9
TCH = 252
NR = TCH // G
UNROLL = 3
NCHUNKS_P = NSUB * TCH
EP = NCHUNKS_P * CHUNK
NP = 10240
RPT = NP // NSUB

_sc_mesh = plsc.VectorSubcoreMesh(core_axis_name="c", subcore_axis_name="s")


@functools.partial(
    pl.kernel,
    mesh=_sc_mesh,
    out_type=jax.ShapeDtypeStruct((NCORES, NP, CW), jnp.float32),
    scratch_types=[
        pltpu.VMEM((TCH, CHUNK), jnp.int32),
        pltpu.VMEM((TCH, CHUNK), jnp.int32),
        pltpu.VMEM((NBUF, CHUNK, CW), jnp.float32),
        pltpu.VMEM_SHARED((NP, CW), jnp.float32),
        pltpu.SemaphoreType.DMA,
        pltpu.SemaphoreType.DMA,
    ],
    compiler_params=pltpu.CompilerParams(use_tc_tiling_on_sc=False),
)
def _sc_agg(x_hbm, si_hbm, di_hbm, out_hbm, sidx, didx, rows, acc, gsem, ssem):
    c = lax.axis_index("c")
    s = lax.axis_index("s")
    xc = x_hbm.at[c]

    zv = jnp.zeros((16,), jnp.float32)

    def _zrow(i, carry):
        for cb in range(CW // 16):
            rows[0, i, pl.ds(cb * 16, 16)] = zv
        return carry

    lax.fori_loop(0, CHUNK, _zrow, 0)
    for k in range(RPT // CHUNK):
        pltpu.sync_copy(rows.at[0], acc.at[pl.ds(s * RPT + k * CHUNK, CHUNK)])

    pltpu.sync_copy(si_hbm.at[pl.ds(s * TCH, TCH)], sidx)
    pltpu.sync_copy(di_hbm.at[pl.ds(s * TCH, TCH)], didx)

    for j in range(2 * G):
        pltpu.async_copy(xc.at[sidx.at[j]], rows.at[j], gsem)
    plsc.subcore_barrier()

    def _gwait(j, slot):
        pltpu.make_async_copy(xc.at[sidx.at[j]], rows.at[slot], gsem).wait()

    def _swait(j, slot):
        pltpu.make_async_copy(rows.at[slot], acc.at[didx.at[j]], ssem).wait()

    def _outer(k, carry):
        for u in range(UNROLL):
            r = k * UNROLL + u
            s0 = (G * u) % NBUF
            for b in range(G):
                slot = s0 + b
                j = r * G + b
                _gwait(j, slot)
                pltpu.async_copy(rows.at[slot], acc.at[didx.at[j]],
                                 ssem, add=True)
            p0 = (s0 - G) % NBUF
            for b in range(G):
                slot = p0 + b
                if u == 0:
                    @pl.when(k > 0)
                    def _():
                        _swait(0, slot)
                else:
                    _swait(0, slot)
                j_new = (r + 2) * G + b

                @pl.when(r < NR - 2)
                def _():
                    pltpu.async_copy(xc.at[sidx.at[j_new]], rows.at[slot], gsem)
        return carry

    lax.fori_loop(0, NR // UNROLL, _outer, 0)
    for b in range(G):
        slot = ((NR - 1) * G + b) % NBUF
        _swait(0, slot)
    plsc.subcore_barrier()

    pltpu.sync_copy(acc.at[pl.ds(s * RPT, RPT)], out_hbm.at[c, pl.ds(s * RPT, RPT)])


def _pad_edges(e):
    src = jnp.concatenate([e[0], jnp.zeros((EP - E,), e.dtype)])
    dst = jnp.concatenate([e[1], jnp.full((EP - E,), N, e.dtype)])
    return (src.astype(jnp.int32).reshape(NCHUNKS_P, CHUNK),
            dst.astype(jnp.int32).reshape(NCHUNKS_P, CHUNK))


def _embed_body(dx_ref, w_ref, b_ref, out_ref):
    out_ref[...] = (
        jnp.dot(dx_ref[...], w_ref[...], preferred_element_type=jnp.float32)
        + b_ref[...]
    )


_embed = pl.pallas_call(
    _embed_body, out_shape=jax.ShapeDtypeStruct((N, H), jnp.float32)
)


def _bn(x, g, bt):
    mu = jnp.mean(x, axis=0, keepdims=True)
    var = jnp.mean((x - mu) ** 2, axis=0, keepdims=True)
    return g * (x - mu) * lax.rsqrt(var + 1e-5) + bt


def _layer_body(x_ref, agg_ref, eps_ref, w1_ref, b1_ref, g1_ref,
                bt1_ref, w2_ref, b2_ref, g2_ref, bt2_ref, out_ref):
    h = (1.0 + eps_ref[0, 0]) * x_ref[...] + agg_ref[...]
    t1 = jnp.dot(h, w1_ref[...], preferred_element_type=jnp.float32) + b1_ref[...]
    t1 = jax.nn.relu(_bn(t1, g1_ref[...], bt1_ref[...]))
    t2 = jnp.dot(t1, w2_ref[...], preferred_element_type=jnp.float32) + b2_ref[...]
    out_ref[...] = jax.nn.relu(_bn(t2, g2_ref[...], bt2_ref[...]))


_layer = pl.pallas_call(
    _layer_body, out_shape=jax.ShapeDtypeStruct((N, H), jnp.float32)
)


def _pool_body(x_ref, bf_ref, out_ref):
    io = lax.broadcasted_iota(jnp.int32, (B, N), 0).astype(jnp.float32)
    p = jnp.where(io == bf_ref[...], 1.0, 0.0)
    sums = jnp.dot(p, x_ref[...], preferred_element_type=jnp.float32)
    cnt = jnp.sum(p, axis=1, keepdims=True)
    out_ref[...] = sums / jnp.maximum(cnt, 1.0)


_pool = pl.pallas_call(
    _pool_body, out_shape=jax.ShapeDtypeStruct((B, H), jnp.float32)
)


def _ln(x, g, bt):
    mu = jnp.mean(x, axis=-1, keepdims=True)
    var = jnp.mean((x - mu) ** 2, axis=-1, keepdims=True)
    return g * (x - mu) * lax.rsqrt(var + 1e-5) + bt


def _fusion_body(aq_ref, bs_ref, sol_ref, t_ref,
                 tw1_ref, tb1_ref, tw2_ref, tb2_ref,
                 iwb_ref, iws_ref, ib_ref, ig_ref, ibt_ref,
                 fw1a_ref, fw1i_ref, fw1t_ref, fb1_ref, fg1_ref, fbt1_ref,
                 fw2_ref, fb2_ref, fg2_ref, fbt2_ref,
                 fw3_ref, fb3_ref, out_ref):
    inter = (
        jnp.dot(bs_ref[...], iwb_ref[...], preferred_element_type=jnp.float32)
        + jnp.dot(sol_ref[...], iws_ref[...], preferred_element_type=jnp.float32)
        + ib_ref[...]
    )
    inter = jax.nn.relu(_ln(inter, ig_ref[...], ibt_ref[...]))
    te = jax.nn.relu(t_ref[...] * tw1_ref[...] + tb1_ref[...])
    te = jnp.dot(te, tw2_ref[...], preferred_element_type=jnp.float32) + tb2_ref[...]
    h = (
        jnp.dot(aq_ref[...], fw1a_ref[...], preferred_element_type=jnp.float32)
        + jnp.dot(inter, fw1i_ref[...], preferred_element_type=jnp.float32)
        + jnp.dot(te, fw1t_ref[...], preferred_element_type=jnp.float32)
        + fb1_ref[...]
    )
    h = jax.nn.relu(_ln(h, fg1_ref[...], fbt1_ref[...]))
    h = jnp.dot(h, fw2_ref[...], preferred_element_type=jnp.float32) + fb2_ref[...]
    h = jax.nn.relu(_ln(h, fg2_ref[...], fbt2_ref[...]))
    out_ref[...] = (
        jnp.sum(h * fw3_ref[...], axis=1, keepdims=True) + fb3_ref[0, 0]
    )


_fusion = pl.pallas_call(
    _fusion_body, out_shape=jax.ShapeDtypeStruct((B, 1), jnp.float32)
)


def _encoder(x0, pk2, p):
    x = _embed(x0, p['emb_W'], p['emb_b'].reshape(1, H))
    for lp in p['layers']:
        xs = jnp.stack([x[:, :CW], x[:, CW:]])
        parts = _sc_agg(xs, pk2[0], pk2[1])
        agg = jnp.concatenate([parts[0, :N], parts[1, :N]], axis=1)
        x = _layer(
            x, agg, lp['eps'].reshape(1, 1),
            lp['W1'], lp['b1'].reshape(1, 2 * H), lp['g1'].reshape(1, 2 * H),
            lp['bt1'].reshape(1, 2 * H),
            lp['W2'], lp['b2'].reshape(1, H), lp['g2'].reshape(1, H),
            lp['bt2'].reshape(1, H),
        )
    return x


def kernel(dx, de, db, sx, se, sb, t, params):
    pk_d = _pad_edges(de)
    pk_s = _pad_edges(se)
    dbf = db.astype(jnp.float32).reshape(1, N)
    sbf = sb.astype(jnp.float32).reshape(1, N)

    x_aq = _encoder(dx, pk_d, params['aq'])
    x_bs = _encoder(dx, pk_d, params['bs'])
    x_sol = _encoder(sx, pk_s, params['sol'])

    emb_aq = _pool(x_aq, dbf)
    emb_bs = _pool(x_bs, dbf)
    emb_sol = _pool(x_sol, sbf)

    tp, ip, fp = params['temp'], params['inter'], params['fus']
    return _fusion(
        emb_aq, emb_bs, emb_sol, t,
        tp['W1'], tp['b1'].reshape(1, 32), tp['W2'], tp['b2'].reshape(1, 32),
        ip['W'][:H], ip['W'][H:], ip['b'].reshape(1, H),
        ip['g'].reshape(1, H), ip['bt'].reshape(1, H),
        fp['W1'][:H], fp['W1'][H:2 * H], fp['W1'][2 * H:],
        fp['b1'].reshape(1, H), fp['g1'].reshape(1, H), fp['bt1'].reshape(1, H),
        fp['W2'], fp['b2'].reshape(1, H // 2), fp['g2'].reshape(1, H // 2),
        fp['bt2'].reshape(1, H // 2),
        fp['W3'].reshape(1, H // 2), fp['b3'].reshape(1, 1),
    )
